# baseline (speedup 1.0000x reference)
import functools

import jax
import jax.numpy as jnp
from jax.experimental import pallas as pl
from jax.experimental.pallas import tpu as pltpu

_R = 1.0 / 7.0


def _tile_extract(tile, off):
    lane = jax.lax.broadcasted_iota(jnp.int32, tile.shape, 1)
    sel = jnp.where(lane == off, tile.astype(jnp.float32), 0.0)
    return jnp.sum(sel, axis=1, keepdims=True)


def _fps_kernel(
    m, n, c, b, v2_ref, norms_ref, xf_ref, out_ref, vbf_ref, *rest
):
    dist_refs = rest[:b]
    sem = rest[b]
    vbf_ref[...] = v2_ref[...].astype(jnp.float32).astype(jnp.bfloat16)
    iota = jax.lax.broadcasted_iota(jnp.int32, (1, n), 1)
    for i in range(b):
        dist_refs[i][...] = jnp.full((1, n), jnp.inf, jnp.float32)

    def start_row_copy(i, src_row, dst_row):
        pltpu.make_async_copy(
            xf_ref.at[i, pl.ds(src_row, 1), :],
            out_ref.at[i, pl.ds(dst_row, 1), :],
            sem.at[i],
        ).start()

    def wait_row_copy(i):
        pltpu.make_async_copy(
            xf_ref.at[i, pl.ds(0, 1), :],
            out_ref.at[i, pl.ds(0, 1), :],
            sem.at[i],
        ).wait()

    cols0 = [vbf_ref[i * c : (i + 1) * c, 0:1] for i in range(b)]
    ynorms0 = [norms_ref[i : i + 1, 0:1] for i in range(b)]
    for i in range(b):
        start_row_copy(i, 0, 0)

    def body(j, carry):
        cols, ynorms = carry

        new_cols, new_ynorms = [], []
        for i in range(b):
            dotv = jax.lax.dot_general(
                cols[i],
                vbf_ref[i * c : (i + 1) * c, :],
                dimension_numbers=(((0,), (0,)), ((), ())),
                preferred_element_type=jnp.float32,
            )
            d = (norms_ref[i : i + 1, :] + ynorms[i]) - 2.0 * dotv
            dist = jnp.minimum(dist_refs[i][...], d)
            dist_refs[i][...] = dist
            mx = jnp.max(dist)
            new = jnp.min(jnp.where(dist == mx, iota, n)).astype(jnp.int32)
            wait_row_copy(i)
            start_row_copy(i, new, j)
            base = pl.multiple_of((new // 128) * 128, 128)
            off = new - base
            vt = vbf_ref[i * c : (i + 1) * c, pl.ds(base, 128)]
            new_cols.append(_tile_extract(vt, off).astype(jnp.bfloat16))
            nt = norms_ref[i : i + 1, pl.ds(base, 128)]
            new_ynorms.append(_tile_extract(nt, off))
        return new_cols, new_ynorms

    jax.lax.fori_loop(1, m, body, (cols0, ynorms0), unroll=2)
    for i in range(b):
        wait_row_copy(i)


def kernel(x):
    b, c, h, w = x.shape
    n = h * w
    mh, mw = round(h * _R), round(w * _R)
    m = max(min(mh * mw, n), 1)
    x_flat = jnp.transpose(x, (0, 2, 3, 1)).reshape(b, n, c)
    norms = jnp.sum(x_flat * x_flat, axis=2)
    v2 = x.reshape(b * c, n).astype(jnp.bfloat16)
    out = pl.pallas_call(
        functools.partial(_fps_kernel, m, n, c, b),
        in_specs=[
            pl.BlockSpec(memory_space=pltpu.MemorySpace.VMEM),
            pl.BlockSpec(memory_space=pltpu.MemorySpace.VMEM),
            pl.BlockSpec(memory_space=pltpu.MemorySpace.HBM),
        ],
        out_specs=pl.BlockSpec(memory_space=pltpu.MemorySpace.VMEM),
        out_shape=jax.ShapeDtypeStruct((b, m, c), x.dtype),
        scratch_shapes=[pltpu.VMEM((b * c, n), jnp.bfloat16)]
        + [pltpu.VMEM((1, n), jnp.float32) for _ in range(b)]
        + [pltpu.SemaphoreType.DMA((b,))],
    )(v2, norms, x_flat)
    return jnp.transpose(out.reshape(b, mh, mw, c), (0, 3, 1, 2))


# speedup vs baseline: 1.1833x; 1.1833x over previous
import functools

import jax
import jax.numpy as jnp
from jax.experimental import pallas as pl
from jax.experimental.pallas import tpu as pltpu

_R = 1.0 / 7.0
_CHUNKS = 4


def _tile_extract(tile, off):
    lane = jax.lax.broadcasted_iota(jnp.int32, tile.shape, 1)
    sel = jnp.where(lane == off, tile.astype(jnp.float32), 0.0)
    return jnp.sum(sel, axis=1, keepdims=True)


def _fps2_kernel(
    m, n, c, b, v2_ref, norms_ref, xf_ref, out_ref, vbf_ref, dist_ref, sem
):
    bc = b * c
    nq = n // _CHUNKS
    vbf_ref[...] = v2_ref[...].astype(jnp.float32).astype(jnp.bfloat16)
    dist_ref[...] = jnp.full((b, n), jnp.inf, jnp.float32)
    rows = jax.lax.broadcasted_iota(jnp.int32, (bc, 1), 0)
    zero = jnp.zeros((bc, 1), jnp.bfloat16)

    def masked_lhs(cols):
        parts = [
            jnp.where((rows >= i * c) & (rows < (i + 1) * c), cols[i], zero)
            for i in range(b)
        ]
        return jnp.concatenate(parts, axis=1)

    def start_row_copy(i, src_row, dst_row):
        pltpu.make_async_copy(
            xf_ref.at[i, pl.ds(src_row, 1), :],
            out_ref.at[i, pl.ds(dst_row, 1), :],
            sem.at[i],
        ).start()

    def wait_row_copy(i):
        pltpu.make_async_copy(
            xf_ref.at[i, pl.ds(0, 1), :],
            out_ref.at[i, pl.ds(0, 1), :],
            sem.at[i],
        ).wait()

    col0 = vbf_ref[:, 0:1]
    lhs0 = masked_lhs([col0] * b)
    ynorm0 = norms_ref[:, 0:1]
    for i in range(b):
        start_row_copy(i, 0, 0)

    def body(j, carry):
        lhs, ynorm = carry
        mxs, ixs = [], []
        for k in range(_CHUNKS):
            sl = pl.ds(k * nq, nq)
            dotk = jax.lax.dot_general(
                lhs,
                vbf_ref[:, sl],
                dimension_numbers=(((0,), (0,)), ((), ())),
                preferred_element_type=jnp.float32,
            )
            dk = (norms_ref[:, sl] + ynorm) - 2.0 * dotk
            distk = jnp.minimum(dist_ref[:, sl], dk)
            dist_ref[:, sl] = distk
            mxk = jnp.max(distk, axis=1, keepdims=True)
            iotak = (
                jax.lax.broadcasted_iota(jnp.int32, (b, nq), 1) + k * nq
            )
            ixs.append(
                jnp.min(jnp.where(distk == mxk, iotak, n), axis=1, keepdims=True)
            )
            mxs.append(mxk)
        mx = mxs[0]
        for k in range(1, _CHUNKS):
            mx = jnp.maximum(mx, mxs[k])
        ix = jnp.full((b, 1), n, jnp.int32)
        for k in range(_CHUNKS):
            ix = jnp.minimum(ix, jnp.where(mxs[k] == mx, ixs[k], n))
        cols, ynorms = [], []
        for i in range(b):
            new = ix[i, 0]
            wait_row_copy(i)
            start_row_copy(i, new, j)
            base = pl.multiple_of((new // 128) * 128, 128)
            off = new - base
            vt = vbf_ref[:, pl.ds(base, 128)]
            cols.append(_tile_extract(vt, off).astype(jnp.bfloat16))
            nt = norms_ref[i : i + 1, pl.ds(base, 128)]
            ynorms.append(_tile_extract(nt, off))
        return masked_lhs(cols), jnp.concatenate(ynorms, axis=0)

    jax.lax.fori_loop(1, m, body, (lhs0, ynorm0))
    for i in range(b):
        wait_row_copy(i)


def kernel(x):
    b, c, h, w = x.shape
    n = h * w
    mh, mw = round(h * _R), round(w * _R)
    m = max(min(mh * mw, n), 1)
    x_flat = jnp.transpose(x, (0, 2, 3, 1)).reshape(b, n, c)
    norms = jnp.sum(x_flat * x_flat, axis=2)
    v2 = x.reshape(b * c, n).astype(jnp.bfloat16)
    out = pl.pallas_call(
        functools.partial(_fps2_kernel, m, n, c, b),
        in_specs=[
            pl.BlockSpec(memory_space=pltpu.MemorySpace.VMEM),
            pl.BlockSpec(memory_space=pltpu.MemorySpace.VMEM),
            pl.BlockSpec(memory_space=pltpu.MemorySpace.HBM),
        ],
        out_specs=pl.BlockSpec(memory_space=pltpu.MemorySpace.VMEM),
        out_shape=jax.ShapeDtypeStruct((b, m, c), x.dtype),
        scratch_shapes=[
            pltpu.VMEM((b * c, n), jnp.bfloat16),
            pltpu.VMEM((b, n), jnp.float32),
            pltpu.SemaphoreType.DMA((b,)),
        ],
    )(v2, norms, x_flat)
    return jnp.transpose(out.reshape(b, mh, mw, c), (0, 3, 1, 2))
